# baseline (speedup 1.0000x reference)
import functools

import jax
import jax.numpy as jnp
from jax import lax
from jax.experimental import pallas as pl
from jax.experimental.pallas import tpu as pltpu
from jax.experimental.pallas import tpu_sc as plsc

_NC = 2
_NS = 16
_NW = _NC * _NS

_BLK = 1024
_CHUNK = 128


def _sc_mesh():
  return plsc.VectorSubcoreMesh(core_axis_name="c", subcore_axis_name="s",
                                num_cores=_NC, num_subcores=_NS)


def _slab_base(c, s, CH0, CH1):
  return jnp.where(c == 0, s * CH0, _NS * CH0 + s * CH1)


_IB = 8


def _make_conv_kernel(NP, H, CH0, CH1, rows_per_sub):
  @functools.partial(
      pl.kernel,
      mesh=_sc_mesh(),
      out_type=jax.ShapeDtypeStruct((_NC, NP, H), jnp.float32),
      scratch_types=[
          pltpu.VMEM((2, _IB, _CHUNK), jnp.int32),
          pltpu.VMEM((2, _IB, _CHUNK), jnp.int32),
          pltpu.VMEM((2, _CHUNK, H), jnp.float32),
          pltpu.SemaphoreType.DMA,
          pltpu.SemaphoreType.DMA,
          pltpu.VMEM_SHARED((NP, H), jnp.float32),
      ],
  )
  def conv_kernel(g, src2d, dst2d, zrows, out, isrc_v, idst_v, rows_v, sem_g,
                  sem_i, acc):
    c = lax.axis_index("c")
    s = lax.axis_index("s")
    my_ch = jnp.where(c == 0, CH0, CH1)
    my_nb = my_ch // _IB
    base = _slab_base(c, s, CH0, CH1)
    pltpu.sync_copy(zrows, acc.at[pl.ds(s * rows_per_sub, rows_per_sub)])

    @pl.when(my_nb > 0)
    def _():
      pltpu.sync_copy(src2d.at[pl.ds(base, _IB)], isrc_v.at[0])
      pltpu.sync_copy(dst2d.at[pl.ds(base, _IB)], idst_v.at[0])

    plsc.subcore_barrier()

    @pl.when(my_nb > 0)
    def _():
      pltpu.async_copy(g.at[isrc_v.at[0, 0]], rows_v.at[0], sem_g)

    def outer(k, carry):
      kb = lax.rem(k, 2)
      nb = 1 - kb
      noff = base + (k + 1) * _IB

      @pl.when(k + 1 < my_nb)
      def _():
        pltpu.async_copy(src2d.at[pl.ds(noff, _IB)], isrc_v.at[nb], sem_i)
        pltpu.async_copy(dst2d.at[pl.ds(noff, _IB)], idst_v.at[nb], sem_i)

      for j in range(_IB):
        rb = j % 2
        pltpu.make_async_copy(g.at[isrc_v.at[kb, j]], rows_v.at[rb],
                              sem_g).wait()
        if j + 1 < _IB:
          pltpu.async_copy(g.at[isrc_v.at[kb, j + 1]], rows_v.at[1 - rb],
                           sem_g)
        else:

          @pl.when(k + 1 < my_nb)
          def _():
            pltpu.make_async_copy(src2d.at[pl.ds(noff, _IB)], isrc_v.at[nb],
                                  sem_i).wait()
            pltpu.make_async_copy(dst2d.at[pl.ds(noff, _IB)], idst_v.at[nb],
                                  sem_i).wait()
            pltpu.async_copy(g.at[isrc_v.at[nb, 0]], rows_v.at[1 - rb],
                             sem_g)

        pltpu.sync_copy(rows_v.at[rb], acc.at[idst_v.at[kb, j]], add=True)
      return carry

    lax.fori_loop(0, my_nb, outer, 0)
    plsc.subcore_barrier()
    sl = pl.ds(s * rows_per_sub, rows_per_sub)
    pltpu.sync_copy(acc.at[sl], out.at[c, sl])

  return conv_kernel


def _make_count_kernel(NP, H, CH0, CH1, rows_per_sub):
  CHM = max(CH0, CH1)

  @functools.partial(
      pl.kernel,
      mesh=_sc_mesh(),
      out_type=jax.ShapeDtypeStruct((_NC, NP, H), jnp.float32),
      scratch_types=[
          pltpu.VMEM((CHM, _CHUNK), jnp.int32),
          pltpu.VMEM((_CHUNK, H), jnp.float32),
          pltpu.VMEM_SHARED((NP, H), jnp.float32),
      ],
  )
  def count_kernel(ones_hbm, dst2d, zrows, out, idst_v, ones_v, acc):
    c = lax.axis_index("c")
    s = lax.axis_index("s")
    my_ch = jnp.where(c == 0, CH0, CH1)
    base = _slab_base(c, s, CH0, CH1)
    pltpu.sync_copy(zrows, acc.at[pl.ds(s * rows_per_sub, rows_per_sub)])
    pltpu.sync_copy(ones_hbm, ones_v)
    pltpu.sync_copy(dst2d.at[pl.ds(base, CHM)], idst_v)
    plsc.subcore_barrier()

    def body(i, carry):
      @pl.when(i < my_ch)
      def _():
        pltpu.sync_copy(ones_v, acc.at[idst_v.at[i]], add=True)
      return carry

    lax.fori_loop(0, CHM, body, 0)
    plsc.subcore_barrier()
    sl = pl.ds(s * rows_per_sub, rows_per_sub)
    pltpu.sync_copy(acc.at[sl], out.at[c, sl])

  return count_kernel


def _dinv_from_cnt(c0, c1):
  deg = 1.0 + c0[:, 0:1] + c1[:, 0:1]
  return lax.rsqrt(deg)


def _pre_body(x_ref, w1_ref, c0_ref, c1_ref, g1_ref):
  dinv = _dinv_from_cnt(c0_ref[...], c1_ref[...])
  g1_ref[...] = dinv * jnp.dot(x_ref[...], w1_ref[...],
                               preferred_element_type=jnp.float32)


def _mid_body(N, a0_ref, a1_ref, g1_ref, c0_ref, c1_ref, b1_ref, w2_ref,
              g2_ref):
  i = pl.program_id(0)
  dinv = _dinv_from_cnt(c0_ref[...], c1_ref[...])
  h1 = dinv * (a0_ref[...] + a1_ref[...] + g1_ref[...]) + b1_ref[...]
  h1 = jnp.maximum(h1, 0.0)
  g2 = dinv * jnp.dot(h1, w2_ref[...], preferred_element_type=jnp.float32)
  rows = i * _BLK + lax.broadcasted_iota(jnp.int32, (_BLK, 1), 0)
  g2_ref[...] = jnp.where(rows < N, g2, 0.0)


def _post_body(B, nblocks, a0_ref, a1_ref, g2_ref, c0_ref, c1_ref, b2_ref,
               bi_ref, wh_ref, bh_ref, out_ref, pool_ref):
  i = pl.program_id(0)
  dinv = _dinv_from_cnt(c0_ref[...], c1_ref[...])
  h2 = dinv * (a0_ref[...] + a1_ref[...] + g2_ref[...]) + b2_ref[...]
  bi = bi_ref[...]

  @pl.when(i == 0)
  def _():
    pool_ref[...] = jnp.full_like(pool_ref, -jnp.inf)

  neg = jnp.float32(-jnp.inf)
  for b in range(B):
    m = bi == b
    contrib = jnp.max(jnp.where(m, h2, neg), axis=0)
    pool_ref[b, :] = jnp.maximum(pool_ref[b, :], contrib)

  @pl.when(i == nblocks - 1)
  def _():
    logits = jnp.dot(pool_ref[...], wh_ref[...],
                     preferred_element_type=jnp.float32) + bh_ref[...]
    out_ref[...] = jax.nn.sigmoid(logits)


def kernel(x, edge_index, batch_idx, W1, b1, W2, b2, Wh, bh):
  N, D = x.shape
  H = W1.shape[1]
  C = Wh.shape[1]
  E = edge_index.shape[1]
  B = 64

  NP = ((N + 1 + _BLK - 1) // _BLK) * _BLK
  nblocks = NP // _BLK
  rows_per_sub = NP // _NS
  pair = -(-E // (_CHUNK * _NS))
  pair = ((pair + 7) // 8) * 8
  CW0 = pair
  CW1 = pair - CW0
  CC0 = ((int(pair * 0.55) + 7) // 8) * 8
  CC1 = pair - CC0
  EP = _NS * pair * _CHUNK

  f32 = jnp.float32
  x_pad = jnp.zeros((NP, D), f32).at[:N].set(x)
  epad = jnp.full((EP - E,), N, jnp.int32)
  src2d = jnp.concatenate([edge_index[0], epad]).reshape(EP // _CHUNK, _CHUNK)
  dst2d = jnp.concatenate([edge_index[1], epad]).reshape(EP // _CHUNK, _CHUNK)
  bi_pad = jnp.full((NP, 1), B, jnp.int32).at[:N, 0].set(batch_idx)

  zrows = jnp.zeros((rows_per_sub, H), f32)
  ones128 = jnp.ones((_CHUNK, H), f32)
  b1r = b1.reshape(1, H)
  b2r = b2.reshape(1, H)
  Wh_pad = jnp.zeros((H, 128), f32).at[:, :C].set(Wh)
  bh_pad = jnp.zeros((1, 128), f32).at[0, :C].set(bh)

  conv = _make_conv_kernel(NP, H, CW0, CW1, rows_per_sub)

  cnt = _make_count_kernel(NP, H, CC0, CC1, rows_per_sub)(ones128, dst2d,
                                                          zrows)
  cnt0, cnt1 = cnt[0], cnt[1]

  row_spec = pl.BlockSpec((_BLK, H), lambda i: (i, 0))
  cnt_spec = row_spec
  full_spec = pl.BlockSpec((H, H), lambda i: (0, 0))
  vec_spec = pl.BlockSpec((1, H), lambda i: (0, 0))

  g1 = pl.pallas_call(
      _pre_body,
      grid=(nblocks,),
      in_specs=[row_spec, full_spec, cnt_spec, cnt_spec],
      out_specs=row_spec,
      out_shape=jax.ShapeDtypeStruct((NP, H), f32),
  )(x_pad, W1, cnt0, cnt1)

  acc1 = conv(g1, src2d, dst2d, zrows)

  g2 = pl.pallas_call(
      functools.partial(_mid_body, N),
      grid=(nblocks,),
      in_specs=[row_spec, row_spec, row_spec, cnt_spec, cnt_spec, vec_spec,
                full_spec],
      out_specs=row_spec,
      out_shape=jax.ShapeDtypeStruct((NP, H), f32),
  )(acc1[0], acc1[1], g1, cnt0, cnt1, b1r, W2)

  acc2 = conv(g2, src2d, dst2d, zrows)

  out = pl.pallas_call(
      functools.partial(_post_body, B, nblocks),
      grid=(nblocks,),
      in_specs=[row_spec, row_spec, row_spec, cnt_spec, cnt_spec, vec_spec,
                pl.BlockSpec((_BLK, 1), lambda i: (i, 0)),
                full_spec, vec_spec],
      out_specs=pl.BlockSpec((B, 128), lambda i: (0, 0)),
      out_shape=jax.ShapeDtypeStruct((B, 128), f32),
      scratch_shapes=[pltpu.VMEM((B, 128), f32)],
  )(acc2[0], acc2[1], g2, cnt0, cnt1, b2r, bi_pad, Wh_pad, bh_pad)

  return out[:, :C]


# speedup vs baseline: 1.3054x; 1.3054x over previous
import functools

import jax
import jax.numpy as jnp
from jax import lax
from jax.experimental import pallas as pl
from jax.experimental.pallas import tpu as pltpu
from jax.experimental.pallas import tpu_sc as plsc

_NC = 2
_NS = 16
_NW = _NC * _NS

_BLK = 1024
_CHUNK = 128


def _sc_mesh():
  return plsc.VectorSubcoreMesh(core_axis_name="c", subcore_axis_name="s",
                                num_cores=_NC, num_subcores=_NS)


def _slab_base(c, s, CH0, CH1):
  return jnp.where(c == 0, s * CH0, _NS * CH0 + s * CH1)


_IB = 8


def _make_conv_kernel(NP, H, CH0, CH1, rows_per_sub):
  @functools.partial(
      pl.kernel,
      mesh=_sc_mesh(),
      out_type=jax.ShapeDtypeStruct((_NC, NP, H), jnp.float32),
      scratch_types=[
          pltpu.VMEM((2, _IB, _CHUNK), jnp.int32),
          pltpu.VMEM((2, _IB, _CHUNK), jnp.int32),
          pltpu.VMEM((2, _CHUNK, H), jnp.float32),
          pltpu.SemaphoreType.DMA,
          pltpu.SemaphoreType.DMA,
          pltpu.VMEM_SHARED((NP, H), jnp.float32),
      ],
  )
  def conv_kernel(g, src2d, dst2d, zrows, out, isrc_v, idst_v, rows_v, sem_g,
                  sem_i, acc):
    c = lax.axis_index("c")
    s = lax.axis_index("s")
    my_ch = jnp.where(c == 0, CH0, CH1)
    my_nb = my_ch // _IB
    base = _slab_base(c, s, CH0, CH1)
    pltpu.sync_copy(zrows, acc.at[pl.ds(s * rows_per_sub, rows_per_sub)])

    @pl.when(my_nb > 0)
    def _():
      pltpu.sync_copy(src2d.at[pl.ds(base, _IB)], isrc_v.at[0])
      pltpu.sync_copy(dst2d.at[pl.ds(base, _IB)], idst_v.at[0])

    plsc.subcore_barrier()

    @pl.when(my_nb > 0)
    def _():
      pltpu.async_copy(g.at[isrc_v.at[0, 0]], rows_v.at[0], sem_g)

    def outer(k, carry):
      kb = lax.rem(k, 2)
      nb = 1 - kb
      noff = base + (k + 1) * _IB

      @pl.when(k + 1 < my_nb)
      def _():
        pltpu.async_copy(src2d.at[pl.ds(noff, _IB)], isrc_v.at[nb], sem_i)
        pltpu.async_copy(dst2d.at[pl.ds(noff, _IB)], idst_v.at[nb], sem_i)

      for j in range(_IB):
        rb = j % 2
        pltpu.make_async_copy(g.at[isrc_v.at[kb, j]], rows_v.at[rb],
                              sem_g).wait()
        if j + 1 < _IB:
          pltpu.async_copy(g.at[isrc_v.at[kb, j + 1]], rows_v.at[1 - rb],
                           sem_g)
        else:

          @pl.when(k + 1 < my_nb)
          def _():
            pltpu.make_async_copy(src2d.at[pl.ds(noff, _IB)], isrc_v.at[nb],
                                  sem_i).wait()
            pltpu.make_async_copy(dst2d.at[pl.ds(noff, _IB)], idst_v.at[nb],
                                  sem_i).wait()
            pltpu.async_copy(g.at[isrc_v.at[nb, 0]], rows_v.at[1 - rb],
                             sem_g)

        pltpu.sync_copy(rows_v.at[rb], acc.at[idst_v.at[kb, j]], add=True)
      return carry

    lax.fori_loop(0, my_nb, outer, 0)
    plsc.subcore_barrier()
    sl = pl.ds(s * rows_per_sub, rows_per_sub)
    pltpu.sync_copy(acc.at[sl], out.at[c, sl])

  return conv_kernel


def _make_count_kernel(NP, H, CH0, CH1, rows_per_sub):
  CHM = max(CH0, CH1)

  @functools.partial(
      pl.kernel,
      mesh=_sc_mesh(),
      out_type=jax.ShapeDtypeStruct((_NC, NP, H), jnp.float32),
      scratch_types=[
          pltpu.VMEM((CHM, _CHUNK), jnp.int32),
          pltpu.VMEM((_CHUNK, H), jnp.float32),
          pltpu.VMEM_SHARED((NP, H), jnp.float32),
      ],
  )
  def count_kernel(ones_hbm, dst2d, zrows, out, idst_v, ones_v, acc):
    c = lax.axis_index("c")
    s = lax.axis_index("s")
    my_ch = jnp.where(c == 0, CH0, CH1)
    base = _slab_base(c, s, CH0, CH1)
    pltpu.sync_copy(zrows, acc.at[pl.ds(s * rows_per_sub, rows_per_sub)])
    pltpu.sync_copy(ones_hbm, ones_v)
    pltpu.sync_copy(dst2d.at[pl.ds(base, CHM)], idst_v)
    plsc.subcore_barrier()

    def body(i, carry):
      @pl.when(i < my_ch)
      def _():
        pltpu.sync_copy(ones_v, acc.at[idst_v.at[i]], add=True)
      return carry

    lax.fori_loop(0, CHM, body, 0)
    plsc.subcore_barrier()
    sl = pl.ds(s * rows_per_sub, rows_per_sub)
    pltpu.sync_copy(acc.at[sl], out.at[c, sl])

  return count_kernel


def _dinv_from_cnt(c0, c1):
  deg = 1.0 + c0[:, 0:1] + c1[:, 0:1]
  return lax.rsqrt(deg)


def _pre_body(x_ref, w1_ref, c0_ref, c1_ref, g1_ref):
  dinv = _dinv_from_cnt(c0_ref[...], c1_ref[...])
  g1_ref[...] = dinv * jnp.dot(x_ref[...], w1_ref[...],
                               preferred_element_type=jnp.float32)


def _mid_body(N, a0_ref, a1_ref, g1_ref, c0_ref, c1_ref, b1_ref, w2_ref,
              g2_ref):
  i = pl.program_id(0)
  dinv = _dinv_from_cnt(c0_ref[...], c1_ref[...])
  h1 = dinv * (a0_ref[...] + a1_ref[...] + g1_ref[...]) + b1_ref[...]
  h1 = jnp.maximum(h1, 0.0)
  g2 = dinv * jnp.dot(h1, w2_ref[...], preferred_element_type=jnp.float32)
  rows = i * _BLK + lax.broadcasted_iota(jnp.int32, (_BLK, 1), 0)
  g2_ref[...] = jnp.where(rows < N, g2, 0.0)


def _post_body(B, nblocks, a0_ref, a1_ref, g2_ref, c0_ref, c1_ref, b2_ref,
               bi_ref, wh_ref, bh_ref, out_ref, pool_ref):
  i = pl.program_id(0)
  dinv = _dinv_from_cnt(c0_ref[...], c1_ref[...])
  h2 = dinv * (a0_ref[...] + a1_ref[...] + g2_ref[...]) + b2_ref[...]
  bi = bi_ref[...]

  @pl.when(i == 0)
  def _():
    pool_ref[...] = jnp.full_like(pool_ref, -jnp.inf)

  neg = jnp.float32(-jnp.inf)
  for b in range(B):
    m = bi == b
    contrib = jnp.max(jnp.where(m, h2, neg), axis=0)
    pool_ref[b, :] = jnp.maximum(pool_ref[b, :], contrib)

  @pl.when(i == nblocks - 1)
  def _():
    logits = jnp.dot(pool_ref[...], wh_ref[...],
                     preferred_element_type=jnp.float32) + bh_ref[...]
    out_ref[...] = jax.nn.sigmoid(logits)


def kernel(x, edge_index, batch_idx, W1, b1, W2, b2, Wh, bh):
  N, D = x.shape
  H = W1.shape[1]
  C = Wh.shape[1]
  E = edge_index.shape[1]
  B = 64

  NP = ((N + 1 + _BLK - 1) // _BLK) * _BLK
  nblocks = NP // _BLK
  rows_per_sub = NP // _NS
  pair = -(-E // (_CHUNK * _NS))
  pair = ((pair + 7) // 8) * 8
  CW0 = ((int(pair * 0.85) + 7) // 8) * 8
  CW1 = pair - CW0
  CC0 = ((int(pair * 0.55) + 7) // 8) * 8
  CC1 = pair - CC0
  EP = _NS * pair * _CHUNK

  f32 = jnp.float32
  x_pad = jnp.zeros((NP, D), f32).at[:N].set(x)
  epad = jnp.full((EP - E,), N, jnp.int32)
  src2d = jnp.concatenate([edge_index[0], epad]).reshape(EP // _CHUNK, _CHUNK)
  dst2d = jnp.concatenate([edge_index[1], epad]).reshape(EP // _CHUNK, _CHUNK)
  bi_pad = jnp.full((NP, 1), B, jnp.int32).at[:N, 0].set(batch_idx)

  zrows = jnp.zeros((rows_per_sub, H), f32)
  ones128 = jnp.ones((_CHUNK, H), f32)
  b1r = b1.reshape(1, H)
  b2r = b2.reshape(1, H)
  Wh_pad = jnp.zeros((H, 128), f32).at[:, :C].set(Wh)
  bh_pad = jnp.zeros((1, 128), f32).at[0, :C].set(bh)

  conv = _make_conv_kernel(NP, H, CW0, CW1, rows_per_sub)

  cnt = _make_count_kernel(NP, H, CC0, CC1, rows_per_sub)(ones128, dst2d,
                                                          zrows)
  cnt0, cnt1 = cnt[0], cnt[1]

  row_spec = pl.BlockSpec((_BLK, H), lambda i: (i, 0))
  cnt_spec = row_spec
  full_spec = pl.BlockSpec((H, H), lambda i: (0, 0))
  vec_spec = pl.BlockSpec((1, H), lambda i: (0, 0))

  g1 = pl.pallas_call(
      _pre_body,
      grid=(nblocks,),
      in_specs=[row_spec, full_spec, cnt_spec, cnt_spec],
      out_specs=row_spec,
      out_shape=jax.ShapeDtypeStruct((NP, H), f32),
  )(x_pad, W1, cnt0, cnt1)

  acc1 = conv(g1, src2d, dst2d, zrows)

  g2 = pl.pallas_call(
      functools.partial(_mid_body, N),
      grid=(nblocks,),
      in_specs=[row_spec, row_spec, row_spec, cnt_spec, cnt_spec, vec_spec,
                full_spec],
      out_specs=row_spec,
      out_shape=jax.ShapeDtypeStruct((NP, H), f32),
  )(acc1[0], acc1[1], g1, cnt0, cnt1, b1r, W2)

  acc2 = conv(g2, src2d, dst2d, zrows)

  out = pl.pallas_call(
      functools.partial(_post_body, B, nblocks),
      grid=(nblocks,),
      in_specs=[row_spec, row_spec, row_spec, cnt_spec, cnt_spec, vec_spec,
                pl.BlockSpec((_BLK, 1), lambda i: (i, 0)),
                full_spec, vec_spec],
      out_specs=pl.BlockSpec((B, 128), lambda i: (0, 0)),
      out_shape=jax.ShapeDtypeStruct((B, 128), f32),
      scratch_shapes=[pltpu.VMEM((B, 128), f32)],
  )(acc2[0], acc2[1], g2, cnt0, cnt1, b2r, bi_pad, Wh_pad, bh_pad)

  return out[:, :C]


# speedup vs baseline: 1.3551x; 1.0381x over previous
import functools

import jax
import jax.numpy as jnp
from jax import lax
from jax.experimental import pallas as pl
from jax.experimental.pallas import tpu as pltpu
from jax.experimental.pallas import tpu_sc as plsc

_NC = 2
_NS = 16
_NW = _NC * _NS

_BLK = 1024
_CHUNK = 128


def _sc_mesh():
  return plsc.VectorSubcoreMesh(core_axis_name="c", subcore_axis_name="s",
                                num_cores=_NC, num_subcores=_NS)


def _slab_base(c, s, CH0, CH1):
  return jnp.where(c == 0, s * CH0, _NS * CH0 + s * CH1)


_IB = 8


def _make_conv_kernel(NP, H, CH0, CH1, rows_per_sub):
  @functools.partial(
      pl.kernel,
      mesh=_sc_mesh(),
      out_type=jax.ShapeDtypeStruct((_NC, NP, H), jnp.float32),
      scratch_types=[
          pltpu.VMEM((2, _IB, _CHUNK), jnp.int32),
          pltpu.VMEM((2, _IB, _CHUNK), jnp.int32),
          pltpu.VMEM((2, _CHUNK, H), jnp.float32),
          pltpu.SemaphoreType.DMA,
          pltpu.SemaphoreType.DMA,
          pltpu.VMEM_SHARED((NP, H), jnp.float32),
      ],
  )
  def conv_kernel(g, src2d, dst2d, zrows, out, isrc_v, idst_v, rows_v, sem_g,
                  sem_i, acc):
    c = lax.axis_index("c")
    s = lax.axis_index("s")
    my_ch = jnp.where(c == 0, CH0, CH1)
    my_nb = my_ch // _IB
    base = _slab_base(c, s, CH0, CH1)
    pltpu.sync_copy(zrows, acc.at[pl.ds(s * rows_per_sub, rows_per_sub)])

    @pl.when(my_nb > 0)
    def _():
      pltpu.sync_copy(src2d.at[pl.ds(base, _IB)], isrc_v.at[0])
      pltpu.sync_copy(dst2d.at[pl.ds(base, _IB)], idst_v.at[0])

    plsc.subcore_barrier()

    @pl.when(my_nb > 0)
    def _():
      pltpu.async_copy(g.at[isrc_v.at[0, 0]], rows_v.at[0], sem_g)

    def outer(k, carry):
      kb = lax.rem(k, 2)
      nb = 1 - kb
      noff = base + (k + 1) * _IB

      @pl.when(k + 1 < my_nb)
      def _():
        pltpu.async_copy(src2d.at[pl.ds(noff, _IB)], isrc_v.at[nb], sem_i)
        pltpu.async_copy(dst2d.at[pl.ds(noff, _IB)], idst_v.at[nb], sem_i)

      for j in range(_IB):
        rb = j % 2
        pltpu.make_async_copy(g.at[isrc_v.at[kb, j]], rows_v.at[rb],
                              sem_g).wait()
        if j + 1 < _IB:
          pltpu.async_copy(g.at[isrc_v.at[kb, j + 1]], rows_v.at[1 - rb],
                           sem_g)
        else:

          @pl.when(k + 1 < my_nb)
          def _():
            pltpu.make_async_copy(src2d.at[pl.ds(noff, _IB)], isrc_v.at[nb],
                                  sem_i).wait()
            pltpu.make_async_copy(dst2d.at[pl.ds(noff, _IB)], idst_v.at[nb],
                                  sem_i).wait()
            pltpu.async_copy(g.at[isrc_v.at[nb, 0]], rows_v.at[1 - rb],
                             sem_g)

        pltpu.sync_copy(rows_v.at[rb], acc.at[idst_v.at[kb, j]], add=True)
      return carry

    lax.fori_loop(0, my_nb, outer, 0)
    plsc.subcore_barrier()
    sl = pl.ds(s * rows_per_sub, rows_per_sub)
    pltpu.sync_copy(acc.at[sl], out.at[c, sl])

  return conv_kernel


def _make_count_kernel(NP, H, CH0, CH1, rows_per_sub):
  CHM = max(CH0, CH1)

  @functools.partial(
      pl.kernel,
      mesh=_sc_mesh(),
      out_type=jax.ShapeDtypeStruct((_NC, NP, H), jnp.float32),
      scratch_types=[
          pltpu.VMEM((CHM, _CHUNK), jnp.int32),
          pltpu.VMEM((_CHUNK, H), jnp.float32),
          pltpu.VMEM_SHARED((NP, H), jnp.float32),
      ],
  )
  def count_kernel(ones_hbm, dst2d, zrows, out, idst_v, ones_v, acc):
    c = lax.axis_index("c")
    s = lax.axis_index("s")
    my_ch = jnp.where(c == 0, CH0, CH1)
    base = _slab_base(c, s, CH0, CH1)
    pltpu.sync_copy(zrows, acc.at[pl.ds(s * rows_per_sub, rows_per_sub)])
    pltpu.sync_copy(ones_hbm, ones_v)
    pltpu.sync_copy(dst2d.at[pl.ds(base, CHM)], idst_v)
    plsc.subcore_barrier()

    def body(i, carry):
      @pl.when(i < my_ch)
      def _():
        pltpu.sync_copy(ones_v, acc.at[idst_v.at[i]], add=True)
      return carry

    lax.fori_loop(0, CHM, body, 0)
    plsc.subcore_barrier()
    sl = pl.ds(s * rows_per_sub, rows_per_sub)
    pltpu.sync_copy(acc.at[sl], out.at[c, sl])

  return count_kernel


def _dinv_from_cnt(c0, c1):
  deg = 1.0 + c0[:, 0:1] + c1[:, 0:1]
  return lax.rsqrt(deg)


def _pre_body(x_ref, w1_ref, c0_ref, c1_ref, g1_ref):
  dinv = _dinv_from_cnt(c0_ref[...], c1_ref[...])
  g1_ref[...] = dinv * jnp.dot(x_ref[...], w1_ref[...],
                               preferred_element_type=jnp.float32)


def _mid_body(N, a0_ref, a1_ref, g1_ref, c0_ref, c1_ref, b1_ref, w2_ref,
              g2_ref):
  i = pl.program_id(0)
  dinv = _dinv_from_cnt(c0_ref[...], c1_ref[...])
  h1 = dinv * (a0_ref[...] + a1_ref[...] + g1_ref[...]) + b1_ref[...]
  h1 = jnp.maximum(h1, 0.0)
  g2 = dinv * jnp.dot(h1, w2_ref[...], preferred_element_type=jnp.float32)
  rows = i * _BLK + lax.broadcasted_iota(jnp.int32, (_BLK, 1), 0)
  g2_ref[...] = jnp.where(rows < N, g2, 0.0)


def _post_body(B, nblocks, a0_ref, a1_ref, g2_ref, c0_ref, c1_ref, b2_ref,
               bi_ref, wh_ref, bh_ref, out_ref, pool_ref):
  i = pl.program_id(0)
  dinv = _dinv_from_cnt(c0_ref[...], c1_ref[...])
  h2 = dinv * (a0_ref[...] + a1_ref[...] + g2_ref[...]) + b2_ref[...]
  bi = bi_ref[...]

  @pl.when(i == 0)
  def _():
    pool_ref[...] = jnp.full_like(pool_ref, -jnp.inf)

  neg = jnp.float32(-jnp.inf)
  for b in range(B):
    m = bi == b
    contrib = jnp.max(jnp.where(m, h2, neg), axis=0)
    pool_ref[b, :] = jnp.maximum(pool_ref[b, :], contrib)

  @pl.when(i == nblocks - 1)
  def _():
    logits = jnp.dot(pool_ref[...], wh_ref[...],
                     preferred_element_type=jnp.float32) + bh_ref[...]
    out_ref[...] = jax.nn.sigmoid(logits)


def kernel(x, edge_index, batch_idx, W1, b1, W2, b2, Wh, bh):
  N, D = x.shape
  H = W1.shape[1]
  C = Wh.shape[1]
  E = edge_index.shape[1]
  B = 64

  NP = ((N + 1 + _BLK - 1) // _BLK) * _BLK
  nblocks = NP // _BLK
  rows_per_sub = NP // _NS
  pair = -(-E // (_CHUNK * _NS))
  pair = ((pair + 7) // 8) * 8
  CW0 = ((int(pair * 0.90) + 7) // 8) * 8
  CW1 = pair - CW0
  CC0 = ((int(pair * 0.55) + 7) // 8) * 8
  CC1 = pair - CC0
  EP = _NS * pair * _CHUNK

  f32 = jnp.float32
  x_pad = jnp.zeros((NP, D), f32).at[:N].set(x)
  epad = jnp.full((EP - E,), N, jnp.int32)
  src2d = jnp.concatenate([edge_index[0], epad]).reshape(EP // _CHUNK, _CHUNK)
  dst2d = jnp.concatenate([edge_index[1], epad]).reshape(EP // _CHUNK, _CHUNK)
  bi_pad = jnp.full((NP, 1), B, jnp.int32).at[:N, 0].set(batch_idx)

  zrows = jnp.zeros((rows_per_sub, H), f32)
  ones128 = jnp.ones((_CHUNK, H), f32)
  b1r = b1.reshape(1, H)
  b2r = b2.reshape(1, H)
  Wh_pad = jnp.zeros((H, 128), f32).at[:, :C].set(Wh)
  bh_pad = jnp.zeros((1, 128), f32).at[0, :C].set(bh)

  conv = _make_conv_kernel(NP, H, CW0, CW1, rows_per_sub)

  cnt = _make_count_kernel(NP, H, CC0, CC1, rows_per_sub)(ones128, dst2d,
                                                          zrows)
  cnt0, cnt1 = cnt[0], cnt[1]

  row_spec = pl.BlockSpec((_BLK, H), lambda i: (i, 0))
  cnt_spec = row_spec
  full_spec = pl.BlockSpec((H, H), lambda i: (0, 0))
  vec_spec = pl.BlockSpec((1, H), lambda i: (0, 0))

  g1 = pl.pallas_call(
      _pre_body,
      grid=(nblocks,),
      in_specs=[row_spec, full_spec, cnt_spec, cnt_spec],
      out_specs=row_spec,
      out_shape=jax.ShapeDtypeStruct((NP, H), f32),
  )(x_pad, W1, cnt0, cnt1)

  acc1 = conv(g1, src2d, dst2d, zrows)

  g2 = pl.pallas_call(
      functools.partial(_mid_body, N),
      grid=(nblocks,),
      in_specs=[row_spec, row_spec, row_spec, cnt_spec, cnt_spec, vec_spec,
                full_spec],
      out_specs=row_spec,
      out_shape=jax.ShapeDtypeStruct((NP, H), f32),
  )(acc1[0], acc1[1], g1, cnt0, cnt1, b1r, W2)

  acc2 = conv(g2, src2d, dst2d, zrows)

  out = pl.pallas_call(
      functools.partial(_post_body, B, nblocks),
      grid=(nblocks,),
      in_specs=[row_spec, row_spec, row_spec, cnt_spec, cnt_spec, vec_spec,
                pl.BlockSpec((_BLK, 1), lambda i: (i, 0)),
                full_spec, vec_spec],
      out_specs=pl.BlockSpec((B, 128), lambda i: (0, 0)),
      out_shape=jax.ShapeDtypeStruct((B, 128), f32),
      scratch_shapes=[pltpu.VMEM((B, 128), f32)],
  )(acc2[0], acc2[1], g2, cnt0, cnt1, b2r, bi_pad, Wh_pad, bh_pad)

  return out[:, :C]


# speedup vs baseline: 1.4552x; 1.0739x over previous
import functools

import jax
import jax.numpy as jnp
from jax import lax
from jax.experimental import pallas as pl
from jax.experimental.pallas import tpu as pltpu
from jax.experimental.pallas import tpu_sc as plsc

_NC = 2
_NS = 16
_NW = _NC * _NS

_BLK = 1024
_CHUNK = 128


def _sc_mesh():
  return plsc.VectorSubcoreMesh(core_axis_name="c", subcore_axis_name="s",
                                num_cores=_NC, num_subcores=_NS)


def _slab_base(c, s, CH0, CH1):
  return jnp.where(c == 0, s * CH0, _NS * CH0 + s * CH1)


_IB = 8


def _make_conv_kernel(NP, H, CH0, CH1, rows_per_sub):
  @functools.partial(
      pl.kernel,
      mesh=_sc_mesh(),
      out_type=jax.ShapeDtypeStruct((_NC, NP, H), jnp.float32),
      scratch_types=[
          pltpu.VMEM((2, _IB, _CHUNK), jnp.int32),
          pltpu.VMEM((2, _IB, _CHUNK), jnp.int32),
          pltpu.VMEM((2, _CHUNK, H), jnp.float32),
          pltpu.SemaphoreType.DMA,
          pltpu.SemaphoreType.DMA,
          pltpu.VMEM_SHARED((NP, H), jnp.float32),
      ],
  )
  def conv_kernel(g, src2d, dst2d, zrows, out, isrc_v, idst_v, rows_v, sem_g,
                  sem_i, acc):
    c = lax.axis_index("c")
    s = lax.axis_index("s")
    my_ch = jnp.where(c == 0, CH0, CH1)
    my_nb = my_ch // _IB
    base = _slab_base(c, s, CH0, CH1)
    pltpu.sync_copy(zrows, acc.at[pl.ds(s * rows_per_sub, rows_per_sub)])

    @pl.when(my_nb > 0)
    def _():
      pltpu.sync_copy(src2d.at[pl.ds(base, _IB)], isrc_v.at[0])
      pltpu.sync_copy(dst2d.at[pl.ds(base, _IB)], idst_v.at[0])

    plsc.subcore_barrier()

    @pl.when(my_nb > 0)
    def _():
      pltpu.async_copy(g.at[isrc_v.at[0, 0]], rows_v.at[0], sem_g)

    def outer(k, carry):
      kb = lax.rem(k, 2)
      nb = 1 - kb
      noff = base + (k + 1) * _IB

      @pl.when(k + 1 < my_nb)
      def _():
        pltpu.async_copy(src2d.at[pl.ds(noff, _IB)], isrc_v.at[nb], sem_i)
        pltpu.async_copy(dst2d.at[pl.ds(noff, _IB)], idst_v.at[nb], sem_i)

      for j in range(_IB):
        rb = j % 2
        pltpu.make_async_copy(g.at[isrc_v.at[kb, j]], rows_v.at[rb],
                              sem_g).wait()
        if j + 1 < _IB:
          pltpu.async_copy(g.at[isrc_v.at[kb, j + 1]], rows_v.at[1 - rb],
                           sem_g)
        else:

          @pl.when(k + 1 < my_nb)
          def _():
            pltpu.make_async_copy(src2d.at[pl.ds(noff, _IB)], isrc_v.at[nb],
                                  sem_i).wait()
            pltpu.make_async_copy(dst2d.at[pl.ds(noff, _IB)], idst_v.at[nb],
                                  sem_i).wait()
            pltpu.async_copy(g.at[isrc_v.at[nb, 0]], rows_v.at[1 - rb],
                             sem_g)

        pltpu.sync_copy(rows_v.at[rb], acc.at[idst_v.at[kb, j]], add=True)
      return carry

    lax.fori_loop(0, my_nb, outer, 0)
    plsc.subcore_barrier()
    sl = pl.ds(s * rows_per_sub, rows_per_sub)
    pltpu.sync_copy(acc.at[sl], out.at[c, sl])

  return conv_kernel


def _make_count_kernel(NP, H, CH0, CH1, rows_per_sub):
  CHM = max(CH0, CH1)

  @functools.partial(
      pl.kernel,
      mesh=_sc_mesh(),
      out_type=jax.ShapeDtypeStruct((_NC, NP, H), jnp.float32),
      scratch_types=[
          pltpu.VMEM((CHM, _CHUNK), jnp.int32),
          pltpu.VMEM((_CHUNK, H), jnp.float32),
          pltpu.VMEM_SHARED((NP, H), jnp.float32),
      ],
  )
  def count_kernel(ones_hbm, dst2d, zrows, out, idst_v, ones_v, acc):
    c = lax.axis_index("c")
    s = lax.axis_index("s")
    my_ch = jnp.where(c == 0, CH0, CH1)
    base = _slab_base(c, s, CH0, CH1)
    pltpu.sync_copy(zrows, acc.at[pl.ds(s * rows_per_sub, rows_per_sub)])
    pltpu.sync_copy(ones_hbm, ones_v)
    pltpu.sync_copy(dst2d.at[pl.ds(base, CHM)], idst_v)
    plsc.subcore_barrier()

    def body(i, carry):
      @pl.when(i < my_ch)
      def _():
        pltpu.sync_copy(ones_v, acc.at[idst_v.at[i]], add=True)
      return carry

    lax.fori_loop(0, CHM, body, 0)
    plsc.subcore_barrier()
    sl = pl.ds(s * rows_per_sub, rows_per_sub)
    pltpu.sync_copy(acc.at[sl], out.at[c, sl])

  return count_kernel


def _dinv_from_cnt(c0, c1):
  deg = 1.0 + c0[:, 0:1] + c1[:, 0:1]
  return lax.rsqrt(deg)


def _pre_body(x_ref, w1_ref, c0_ref, c1_ref, g1_ref):
  dinv = _dinv_from_cnt(c0_ref[...], c1_ref[...])
  g1_ref[...] = dinv * jnp.dot(x_ref[...], w1_ref[...],
                               preferred_element_type=jnp.float32)


def _mid_body(N, a0_ref, a1_ref, g1_ref, c0_ref, c1_ref, b1_ref, w2_ref,
              g2_ref):
  i = pl.program_id(0)
  dinv = _dinv_from_cnt(c0_ref[...], c1_ref[...])
  h1 = dinv * (a0_ref[...] + a1_ref[...] + g1_ref[...]) + b1_ref[...]
  h1 = jnp.maximum(h1, 0.0)
  g2 = dinv * jnp.dot(h1, w2_ref[...], preferred_element_type=jnp.float32)
  rows = i * _BLK + lax.broadcasted_iota(jnp.int32, (_BLK, 1), 0)
  g2_ref[...] = jnp.where(rows < N, g2, 0.0)


def _post_body(B, nblocks, a0_ref, a1_ref, g2_ref, c0_ref, c1_ref, b2_ref,
               bi_ref, wh_ref, bh_ref, out_ref, pool_ref):
  i = pl.program_id(0)
  dinv = _dinv_from_cnt(c0_ref[...], c1_ref[...])
  h2 = dinv * (a0_ref[...] + a1_ref[...] + g2_ref[...]) + b2_ref[...]
  bi = bi_ref[...]

  @pl.when(i == 0)
  def _():
    pool_ref[...] = jnp.full_like(pool_ref, -jnp.inf)

  neg = jnp.float32(-jnp.inf)
  lo = jnp.min(bi)
  hi = jnp.minimum(jnp.max(bi), B - 1)

  def upd(b, carry):
    m = bi == b
    contrib = jnp.max(jnp.where(m, h2, neg), axis=0)
    pool_ref[b, :] = jnp.maximum(pool_ref[b, :], contrib)
    return carry

  lax.fori_loop(lo, hi + 1, upd, 0)

  @pl.when(i == nblocks - 1)
  def _():
    logits = jnp.dot(pool_ref[...], wh_ref[...],
                     preferred_element_type=jnp.float32) + bh_ref[...]
    out_ref[...] = jax.nn.sigmoid(logits)


def kernel(x, edge_index, batch_idx, W1, b1, W2, b2, Wh, bh):
  N, D = x.shape
  H = W1.shape[1]
  C = Wh.shape[1]
  E = edge_index.shape[1]
  B = 64

  NP = ((N + 1 + _BLK - 1) // _BLK) * _BLK
  nblocks = NP // _BLK
  rows_per_sub = NP // _NS
  pair = -(-E // (_CHUNK * _NS))
  pair = ((pair + 7) // 8) * 8
  CW0 = ((int(pair * 0.90) + 7) // 8) * 8
  CW1 = pair - CW0
  CC0 = ((int(pair * 0.55) + 7) // 8) * 8
  CC1 = pair - CC0
  EP = _NS * pair * _CHUNK

  f32 = jnp.float32
  x_pad = jnp.zeros((NP, D), f32).at[:N].set(x)
  epad = jnp.full((EP - E,), N, jnp.int32)
  src2d = jnp.concatenate([edge_index[0], epad]).reshape(EP // _CHUNK, _CHUNK)
  dst2d = jnp.concatenate([edge_index[1], epad]).reshape(EP // _CHUNK, _CHUNK)
  bi_pad = jnp.full((NP, 1), B, jnp.int32).at[:N, 0].set(batch_idx)

  zrows = jnp.zeros((rows_per_sub, H), f32)
  ones128 = jnp.ones((_CHUNK, H), f32)
  b1r = b1.reshape(1, H)
  b2r = b2.reshape(1, H)
  Wh_pad = jnp.zeros((H, 128), f32).at[:, :C].set(Wh)
  bh_pad = jnp.zeros((1, 128), f32).at[0, :C].set(bh)

  conv = _make_conv_kernel(NP, H, CW0, CW1, rows_per_sub)

  cnt = _make_count_kernel(NP, H, CC0, CC1, rows_per_sub)(ones128, dst2d,
                                                          zrows)
  cnt0, cnt1 = cnt[0], cnt[1]

  row_spec = pl.BlockSpec((_BLK, H), lambda i: (i, 0))
  cnt_spec = row_spec
  full_spec = pl.BlockSpec((H, H), lambda i: (0, 0))
  vec_spec = pl.BlockSpec((1, H), lambda i: (0, 0))

  g1 = pl.pallas_call(
      _pre_body,
      grid=(nblocks,),
      in_specs=[row_spec, full_spec, cnt_spec, cnt_spec],
      out_specs=row_spec,
      out_shape=jax.ShapeDtypeStruct((NP, H), f32),
  )(x_pad, W1, cnt0, cnt1)

  acc1 = conv(g1, src2d, dst2d, zrows)

  g2 = pl.pallas_call(
      functools.partial(_mid_body, N),
      grid=(nblocks,),
      in_specs=[row_spec, row_spec, row_spec, cnt_spec, cnt_spec, vec_spec,
                full_spec],
      out_specs=row_spec,
      out_shape=jax.ShapeDtypeStruct((NP, H), f32),
  )(acc1[0], acc1[1], g1, cnt0, cnt1, b1r, W2)

  acc2 = conv(g2, src2d, dst2d, zrows)

  out = pl.pallas_call(
      functools.partial(_post_body, B, nblocks),
      grid=(nblocks,),
      in_specs=[row_spec, row_spec, row_spec, cnt_spec, cnt_spec, vec_spec,
                pl.BlockSpec((_BLK, 1), lambda i: (i, 0)),
                full_spec, vec_spec],
      out_specs=pl.BlockSpec((B, 128), lambda i: (0, 0)),
      out_shape=jax.ShapeDtypeStruct((B, 128), f32),
      scratch_shapes=[pltpu.VMEM((B, 128), f32)],
  )(acc2[0], acc2[1], g2, cnt0, cnt1, b2r, bi_pad, Wh_pad, bh_pad)

  return out[:, :C]
